# Initial kernel scaffold; baseline (speedup 1.0000x reference)
import jax
import jax.numpy as jnp
from jax.experimental import pallas as pl


def kernel(input, ids, memory, mask):
    raise NotImplementedError("write your pallas kernel here")


# speedup vs baseline: 1086.3179x; 1086.3179x over previous
import math

import jax
import jax.numpy as jnp
from jax import lax
from jax.experimental import pallas as pl
from jax.experimental.pallas import tpu as pltpu
from jax.experimental.pallas import tpu_sc as plsc

DECAY = 0.9
LOG_D = math.log(DECAY)
B = 16384
M = 100000
L = 16
CHUNK = 4096
NCHUNK = B // CHUNK
VPC = CHUNK // L


def _lanes():
    return lax.iota(jnp.int32, L)


def _sc_videosoftmax(x_hbm, ids_hbm, mem_hbm, out_hbm, cnt, idsv, fbuf, xbuf):
    cid = lax.axis_index("c")
    sid = lax.axis_index("s")

    @pl.when(jnp.logical_and(cid == 0, sid == 0))
    def _():
        pltpu.sync_copy(ids_hbm, idsv)
        pltpu.sync_copy(mem_hbm.at[pl.ds(0, M)], cnt)
        lanes = _lanes()

        for t in range(NCHUNK):
            tbase = t * CHUNK

            def stepA(m, _):
                base = tbase + m * L
                v = idsv[pl.ds(base, L)]
                bc = plsc.load_gather(cnt, [v])
                p = jnp.zeros((L,), jnp.float32)
                tot = jnp.zeros((L,), jnp.float32)
                for j in range(L):
                    vj = jnp.full((L,), v[j], jnp.int32)
                    eq = v == vj
                    tot += jnp.where(eq, 1.0, 0.0)
                    p += jnp.where(jnp.logical_and(eq, lanes > j), 1.0, 0.0)
                fbuf[pl.ds(m * L, L)] = bc + p + 1.0
                plsc.store_scatter(cnt, [v], bc + tot)
                return _

            lax.fori_loop(0, VPC, stepA, None)
            pltpu.sync_copy(fbuf, out_hbm.at[pl.ds(tbase, CHUNK)])

        for t in range(NCHUNK):
            tbase = t * CHUNK
            pltpu.sync_copy(out_hbm.at[pl.ds(tbase, CHUNK)], fbuf)
            pltpu.sync_copy(x_hbm.at[pl.ds(tbase, CHUNK)], xbuf)

            def stepB(m, _):
                base = tbase + m * L
                v = idsv[pl.ds(base, L)]
                r = fbuf[pl.ds(m * L, L)]
                k = plsc.load_gather(cnt, [v])
                xv = xbuf[pl.ds(m * L, L)]
                f = jnp.where(r <= 1.0, 1.0, 1.0 - DECAY)
                fbuf[pl.ds(m * L, L)] = jnp.exp(xv + (k - r) * LOG_D) * f
                return _

            lax.fori_loop(0, VPC, stepB, None)
            pltpu.sync_copy(fbuf, out_hbm.at[pl.ds(tbase, CHUNK)])

        pltpu.sync_copy(mem_hbm.at[pl.ds(0, M)], cnt)
        for t in range(NCHUNK):
            tbase = t * CHUNK
            pltpu.sync_copy(out_hbm.at[pl.ds(tbase, CHUNK)], fbuf)

            def stepC(m, _):
                base = tbase + m * L
                v = idsv[pl.ds(base, L)]
                u = fbuf[pl.ds(m * L, L)]
                sb = plsc.load_gather(cnt, [v])
                usum = jnp.zeros((L,), jnp.float32)
                for j in range(L):
                    vj = jnp.full((L,), v[j], jnp.int32)
                    uj = jnp.full((L,), u[j], jnp.float32)
                    usum += jnp.where(v == vj, uj, 0.0)
                plsc.store_scatter(cnt, [v], sb + usum)
                return _

            lax.fori_loop(0, VPC, stepC, None)

        for t in range(NCHUNK):
            tbase = t * CHUNK
            pltpu.sync_copy(x_hbm.at[pl.ds(tbase, CHUNK)], xbuf)

            def stepD(m, _):
                base = tbase + m * L
                v = idsv[pl.ds(base, L)]
                s = plsc.load_gather(cnt, [v])
                xv = xbuf[pl.ds(m * L, L)]
                fbuf[pl.ds(m * L, L)] = jnp.exp(xv) / s
                return _

            lax.fori_loop(0, VPC, stepD, None)
            pltpu.sync_copy(fbuf, out_hbm.at[pl.ds(tbase, CHUNK)])


def kernel(input, ids, memory, mask):
    del mask
    mesh = plsc.VectorSubcoreMesh(
        core_axis_name="c", subcore_axis_name="s", num_cores=2, num_subcores=16
    )
    run = pl.kernel(
        _sc_videosoftmax,
        out_type=jax.ShapeDtypeStruct((B,), jnp.float32),
        mesh=mesh,
        compiler_params=pltpu.CompilerParams(needs_layout_passes=False),
        scratch_types=[
            pltpu.VMEM((M,), jnp.float32),
            pltpu.VMEM((B,), jnp.int32),
            pltpu.VMEM((CHUNK,), jnp.float32),
            pltpu.VMEM((CHUNK,), jnp.float32),
        ],
    )
    return run(input, ids, memory)
